# baseline (speedup 1.0000x reference)
import functools

import jax
import jax.numpy as jnp
from jax import lax
from jax.experimental import pallas as pl
from jax.experimental.pallas import tpu as pltpu
from jax.experimental.pallas import tpu_sc as plsc

_G = 512
_F = 8
_D = 256
_BLK = 512
_NEG = -3.4e38

_NS = 16
_NC = 2
_GP = 520


def _fv_body(x_ref, filw_ref, filb_ref, fv_ref):
    fv = jnp.dot(x_ref[...], filw_ref[...], preferred_element_type=jnp.float32)
    fv_ref[...] = fv + filb_ref[...]


def _out_body(x_ref, fv_ref, death_ref, w1_ref, w2_ref, outb_ref,
              tri_t_ref, gmu_ref, gsig_ref, lw_ref, lb_ref, rhc_ref, rhr_ref,
              out_ref):
    birth = fv_ref[...]
    death = death_ref[...]

    cs = []
    for k in range(3):
        cs.append(jax.nn.relu(death - jnp.abs(tri_t_ref[k] - birth)))
    for k in range(3):
        d2 = (birth - gmu_ref[k, 0]) ** 2 + (death - gmu_ref[k, 1]) ** 2
        cs.append(jnp.exp(-d2 / (2.0 * gsig_ref[k] ** 2)))
    for k in range(3):
        cs.append(birth * lw_ref[0, k] + death * lw_ref[1, k] + lb_ref[k])
    r_abs = jnp.abs(rhr_ref[0])
    for k in range(3):
        l1 = jnp.abs(birth - rhc_ref[k, 0]) + jnp.abs(death - rhc_ref[k, 1])
        cs.append(1.0 / (1.0 + l1) - 1.0 / (1.0 + jnp.abs(r_abs - l1)))
    coord = jnp.concatenate(cs, axis=1)

    out = jnp.dot(x_ref[...], w1_ref[...], preferred_element_type=jnp.float32)
    out = out + jnp.dot(coord, w2_ref[...], preferred_element_type=jnp.float32)
    out_ref[...] = out + outb_ref[...]


def _vperm16(x, idx):
    dn = lax.GatherDimensionNumbers(
        offset_dims=(), collapsed_slice_dims=(0,), start_index_map=(0,))
    return lax.gather(x, idx[:, None], dn, slice_sizes=(1,),
                      mode=lax.GatherScatterMode.PROMISE_IN_BOUNDS)


def _sc_segmax_death(fv_flat, batch_p, n_pad):
    chunk = n_pad // _NS
    pairs = chunk // 2
    dchunk = n_pad // (_NS * _NC)
    dpairs = dchunk // 2
    slc = (_G * _F) // _NS
    mesh = plsc.VectorSubcoreMesh(core_axis_name="c", subcore_axis_name="s")

    @functools.partial(
        pl.kernel, mesh=mesh,
        compiler_params=pltpu.CompilerParams(needs_layout_passes=False),
        out_type=jax.ShapeDtypeStruct((n_pad * _F,), jnp.float32),
        scratch_types=[
            pltpu.VMEM((chunk * _F,), jnp.float32),
            pltpu.VMEM((chunk,), jnp.int32),
            pltpu.VMEM((chunk * _F,), jnp.int32),
            pltpu.VMEM((_GP * _F,), jnp.float32),
            pltpu.VMEM((_NS, slc), jnp.float32),
            pltpu.VMEM((_GP * _F,), jnp.float32),
            pltpu.VMEM((dchunk * _F,), jnp.float32),
            pltpu.VMEM_SHARED((_NS, _GP * _F), jnp.float32),
            pltpu.VMEM_SHARED((_G * _F,), jnp.float32),
        ],
    )
    def sc_kernel(fv_hbm, batch_hbm, death_hbm,
                  fvv, bv, idxv, localv, stagev, finalv, deathv,
                  sh_loc, sh_fin):
        c = lax.axis_index("c")
        s = lax.axis_index("s")
        wid = s * _NC + c

        i16 = lax.iota(jnp.int32, 16)
        node_off = lax.shift_right_logical(i16, 3)
        feat = lax.bitwise_and(i16, 7)
        perm = lax.bitwise_xor(i16, 8)

        neg = jnp.full((16,), _NEG, jnp.float32)

        def init_body(j, _):
            localv[pl.ds(j * 16, 16)] = neg
            return _
        lax.fori_loop(0, (_GP * _F) // 16, init_body, None)

        pltpu.sync_copy(fv_hbm.at[pl.ds(s * chunk * _F, chunk * _F)], fvv)
        pltpu.sync_copy(batch_hbm.at[pl.ds(s * chunk, chunk)], bv)

        def scan_body(i, _):
            base = i * 16
            v = fvv[pl.ds(base, 16)]
            g = plsc.load_gather(bv, [node_off + 2 * i])
            idx = g * 8 + feat
            idxv[pl.ds(base, 16)] = idx
            old = plsc.load_gather(localv, [idx])
            new = jnp.maximum(old, v)
            idx_sw = _vperm16(idx, perm)
            new_sw = _vperm16(new, perm)
            new = jnp.where(idx == idx_sw, jnp.maximum(new, new_sw), new)
            plsc.store_scatter(localv, [idx], new)
            return _
        lax.fori_loop(0, pairs, scan_body, None)

        pltpu.sync_copy(localv, sh_loc.at[s])
        plsc.subcore_barrier()

        pltpu.sync_copy(sh_loc.at[:, pl.ds(s * slc, slc)], stagev)

        def comb_body(j, _):
            m = stagev[0, pl.ds(j * 16, 16)]
            def inner(t, m):
                return jnp.maximum(m, stagev[t, pl.ds(j * 16, 16)])
            m = lax.fori_loop(1, _NS, inner, m)
            finalv[pl.ds(j * 16, 16)] = m
            return _
        lax.fori_loop(0, slc // 16, comb_body, None)
        pltpu.sync_copy(finalv.at[pl.ds(0, slc)],
                        sh_fin.at[pl.ds(s * slc, slc)])
        plsc.subcore_barrier()

        pltpu.sync_copy(sh_fin, finalv.at[pl.ds(0, _G * _F)])
        zero = jnp.zeros((16,), jnp.float32)

        def zpad_body(j, _):
            finalv[pl.ds(_G * _F + j * 16, 16)] = zero
            return _
        lax.fori_loop(0, ((_GP - _G) * _F) // 16, zpad_body, None)

        ibase = c * dchunk * _F

        def death_body(j, _):
            idx = idxv[pl.ds(ibase + j * 16, 16)]
            deathv[pl.ds(j * 16, 16)] = plsc.load_gather(finalv, [idx])
            return _
        lax.fori_loop(0, dpairs, death_body, None)

        pltpu.sync_copy(deathv,
                        death_hbm.at[pl.ds(wid * dchunk * _F, dchunk * _F)])

    return sc_kernel(fv_flat, batch_p)


def kernel(x, batch, fil_W, fil_b, tri_t, gauss_mu, gauss_sigma,
           line_W, line_b, rh_c, rh_r, out_W, out_b):
    n = x.shape[0]
    nblk = (n + _BLK - 1) // _BLK
    n_pad = nblk * _BLK
    x_p = jnp.pad(x, ((0, n_pad - n), (0, 0)))
    batch_p = jnp.pad(batch, (0, n_pad - n), constant_values=_G)

    fv = pl.pallas_call(
        _fv_body,
        grid=(nblk,),
        in_specs=[
            pl.BlockSpec((_BLK, _D), lambda i: (i, 0)),
            pl.BlockSpec((_D, _F), lambda i: (0, 0)),
            pl.BlockSpec((1, _F), lambda i: (0, 0)),
        ],
        out_specs=pl.BlockSpec((_BLK, _F), lambda i: (i, 0)),
        out_shape=jax.ShapeDtypeStruct((n_pad, _F), jnp.float32),
    )(x_p, fil_W, fil_b.reshape(1, _F))

    death = _sc_segmax_death(fv.reshape(-1), batch_p, n_pad)
    death = death.reshape(n_pad, _F)

    w2 = out_W[_D:].reshape(_F, 12, _D).transpose(1, 0, 2).reshape(12 * _F, _D)

    smem = pl.BlockSpec(memory_space=pltpu.SMEM)
    out_p = pl.pallas_call(
        _out_body,
        grid=(nblk,),
        in_specs=[
            pl.BlockSpec((_BLK, _D), lambda i: (i, 0)),
            pl.BlockSpec((_BLK, _F), lambda i: (i, 0)),
            pl.BlockSpec((_BLK, _F), lambda i: (i, 0)),
            pl.BlockSpec((_D, _D), lambda i: (0, 0)),
            pl.BlockSpec((12 * _F, _D), lambda i: (0, 0)),
            pl.BlockSpec((1, _D), lambda i: (0, 0)),
            smem, smem, smem, smem, smem, smem, smem,
        ],
        out_specs=pl.BlockSpec((_BLK, _D), lambda i: (i, 0)),
        out_shape=jax.ShapeDtypeStruct((n_pad, _D), jnp.float32),
    )(x_p, fv, death, out_W[:_D], w2, out_b.reshape(1, _D),
      tri_t, gauss_mu, gauss_sigma, line_W, line_b, rh_c, rh_r)

    return out_p[:n]


# speedup vs baseline: 1.2157x; 1.2157x over previous
import functools

import jax
import jax.numpy as jnp
from jax import lax
from jax.experimental import pallas as pl
from jax.experimental.pallas import tpu as pltpu
from jax.experimental.pallas import tpu_sc as plsc

_G = 512
_F = 8
_D = 256
_BLK = 512
_NEG = -3.4e38

_NS = 16
_NC = 2
_GP = 520


def _fv_body(x_ref, filw_ref, filb_ref, fv_ref):
    fv = jnp.dot(x_ref[...], filw_ref[...], preferred_element_type=jnp.float32)
    fv_ref[...] = fv + filb_ref[...]


def _out_body(x_ref, fv_ref, death_ref, w1_ref, w2_ref, outb_ref,
              tri_t_ref, gmu_ref, gsig_ref, lw_ref, lb_ref, rhc_ref, rhr_ref,
              out_ref):
    birth = fv_ref[...]
    death = death_ref[...]

    cs = []
    for k in range(3):
        cs.append(jax.nn.relu(death - jnp.abs(tri_t_ref[k] - birth)))
    for k in range(3):
        d2 = (birth - gmu_ref[k, 0]) ** 2 + (death - gmu_ref[k, 1]) ** 2
        cs.append(jnp.exp(-d2 / (2.0 * gsig_ref[k] ** 2)))
    for k in range(3):
        cs.append(birth * lw_ref[0, k] + death * lw_ref[1, k] + lb_ref[k])
    r_abs = jnp.abs(rhr_ref[0])
    for k in range(3):
        l1 = jnp.abs(birth - rhc_ref[k, 0]) + jnp.abs(death - rhc_ref[k, 1])
        cs.append(1.0 / (1.0 + l1) - 1.0 / (1.0 + jnp.abs(r_abs - l1)))
    coord = jnp.concatenate(cs, axis=1)

    out = jnp.dot(x_ref[...], w1_ref[...], preferred_element_type=jnp.float32)
    out = out + jnp.dot(coord, w2_ref[...], preferred_element_type=jnp.float32)
    out_ref[...] = out + outb_ref[...]


def _vperm16(x, idx):
    dn = lax.GatherDimensionNumbers(
        offset_dims=(), collapsed_slice_dims=(0,), start_index_map=(0,))
    return lax.gather(x, idx[:, None], dn, slice_sizes=(1,),
                      mode=lax.GatherScatterMode.PROMISE_IN_BOUNDS)


def _sc_segmax_death(fv_flat, batch_p, n_pad):
    chunk = n_pad // _NS
    pairs = chunk // 2
    dchunk = n_pad // (_NS * _NC)
    dpairs = dchunk // 2
    slc = (_G * _F) // _NS
    mesh = plsc.VectorSubcoreMesh(core_axis_name="c", subcore_axis_name="s")

    @functools.partial(
        pl.kernel, mesh=mesh,
        compiler_params=pltpu.CompilerParams(needs_layout_passes=False),
        out_type=jax.ShapeDtypeStruct((n_pad * _F,), jnp.float32),
        scratch_types=[
            pltpu.VMEM((chunk * _F,), jnp.float32),
            pltpu.VMEM((chunk,), jnp.int32),
            pltpu.VMEM((chunk * _F,), jnp.int32),
            pltpu.VMEM((_GP * _F,), jnp.float32),
            pltpu.VMEM((_NS, slc), jnp.float32),
            pltpu.VMEM((_GP * _F,), jnp.float32),
            pltpu.VMEM((dchunk * _F,), jnp.float32),
            pltpu.VMEM_SHARED((_NS, _GP * _F), jnp.float32),
            pltpu.VMEM_SHARED((_G * _F,), jnp.float32),
        ],
    )
    def sc_kernel(fv_hbm, batch_hbm, death_hbm,
                  fvv, bv, idxv, localv, stagev, finalv, deathv,
                  sh_loc, sh_fin):
        c = lax.axis_index("c")
        s = lax.axis_index("s")
        wid = s * _NC + c

        i16 = lax.iota(jnp.int32, 16)
        node_off = lax.shift_right_logical(i16, 3)
        feat = lax.bitwise_and(i16, 7)
        perm = lax.bitwise_xor(i16, 8)

        neg = jnp.full((16,), _NEG, jnp.float32)

        def init_body(j, _):
            localv[pl.ds(j * 16, 16)] = neg
            return _
        lax.fori_loop(0, (_GP * _F) // 16, init_body, None)

        pltpu.sync_copy(fv_hbm.at[pl.ds(s * chunk * _F, chunk * _F)], fvv)
        pltpu.sync_copy(batch_hbm.at[pl.ds(s * chunk, chunk)], bv)

        def scan_body(i, _):
            base = i * 16
            v = fvv[pl.ds(base, 16)]
            g = plsc.load_gather(bv, [node_off + 2 * i])
            idx = g * 8 + feat
            idxv[pl.ds(base, 16)] = idx
            old = plsc.load_gather(localv, [idx])
            new = jnp.maximum(old, v)
            idx_sw = _vperm16(idx, perm)
            new_sw = _vperm16(new, perm)
            new = jnp.where(idx == idx_sw, jnp.maximum(new, new_sw), new)
            plsc.store_scatter(localv, [idx], new)
            return _
        lax.fori_loop(0, pairs, scan_body, None)

        pltpu.sync_copy(localv, sh_loc.at[s])
        plsc.subcore_barrier()

        pltpu.sync_copy(sh_loc.at[:, pl.ds(s * slc, slc)], stagev)

        def comb_body(j, _):
            m = stagev[0, pl.ds(j * 16, 16)]
            def inner(t, m):
                return jnp.maximum(m, stagev[t, pl.ds(j * 16, 16)])
            m = lax.fori_loop(1, _NS, inner, m)
            finalv[pl.ds(j * 16, 16)] = m
            return _
        lax.fori_loop(0, slc // 16, comb_body, None)
        pltpu.sync_copy(finalv.at[pl.ds(0, slc)],
                        sh_fin.at[pl.ds(s * slc, slc)])
        plsc.subcore_barrier()

        pltpu.sync_copy(sh_fin, finalv.at[pl.ds(0, _G * _F)])
        zero = jnp.zeros((16,), jnp.float32)

        def zpad_body(j, _):
            finalv[pl.ds(_G * _F + j * 16, 16)] = zero
            return _
        lax.fori_loop(0, ((_GP - _G) * _F) // 16, zpad_body, None)

        ibase = c * dchunk * _F

        def death_body(j, _):
            idx = idxv[pl.ds(ibase + j * 16, 16)]
            deathv[pl.ds(j * 16, 16)] = plsc.load_gather(finalv, [idx])
            return _
        lax.fori_loop(0, dpairs, death_body, None)

        pltpu.sync_copy(deathv,
                        death_hbm.at[pl.ds(wid * dchunk * _F, dchunk * _F)])

    return sc_kernel(fv_flat, batch_p)


def kernel(x, batch, fil_W, fil_b, tri_t, gauss_mu, gauss_sigma,
           line_W, line_b, rh_c, rh_r, out_W, out_b):
    n = x.shape[0]
    nblk = (n + _BLK - 1) // _BLK
    n_pad = nblk * _BLK
    batch_p = jnp.pad(batch, (0, n_pad - n), constant_values=_G)

    fv = pl.pallas_call(
        _fv_body,
        grid=(nblk,),
        in_specs=[
            pl.BlockSpec((_BLK, _D), lambda i: (i, 0)),
            pl.BlockSpec((_D, _F), lambda i: (0, 0)),
            pl.BlockSpec((1, _F), lambda i: (0, 0)),
        ],
        out_specs=pl.BlockSpec((_BLK, _F), lambda i: (i, 0)),
        out_shape=jax.ShapeDtypeStruct((n_pad, _F), jnp.float32),
    )(x, fil_W, fil_b.reshape(1, _F))

    death = _sc_segmax_death(fv.reshape(-1), batch_p, n_pad)
    death = death.reshape(n_pad, _F)

    w2 = out_W[_D:].reshape(_F, 12, _D).transpose(1, 0, 2).reshape(12 * _F, _D)

    smem = pl.BlockSpec(memory_space=pltpu.SMEM)
    out_p = pl.pallas_call(
        _out_body,
        grid=(nblk,),
        in_specs=[
            pl.BlockSpec((_BLK, _D), lambda i: (i, 0)),
            pl.BlockSpec((_BLK, _F), lambda i: (i, 0)),
            pl.BlockSpec((_BLK, _F), lambda i: (i, 0)),
            pl.BlockSpec((_D, _D), lambda i: (0, 0)),
            pl.BlockSpec((12 * _F, _D), lambda i: (0, 0)),
            pl.BlockSpec((1, _D), lambda i: (0, 0)),
            smem, smem, smem, smem, smem, smem, smem,
        ],
        out_specs=pl.BlockSpec((_BLK, _D), lambda i: (i, 0)),
        out_shape=jax.ShapeDtypeStruct((n, _D), jnp.float32),
    )(x, fv, death, out_W[:_D], w2, out_b.reshape(1, _D),
      tri_t, gauss_mu, gauss_sigma, line_W, line_b, rh_c, rh_r)

    return out_p


# speedup vs baseline: 1.3444x; 1.1059x over previous
import functools

import jax
import jax.numpy as jnp
from jax import lax
from jax.experimental import pallas as pl
from jax.experimental.pallas import tpu as pltpu
from jax.experimental.pallas import tpu_sc as plsc

_G = 512
_F = 8
_D = 256
_BLK = 512
_NEG = -3.4e38

_NS = 16
_NC = 2
_GP = 520


def _fv_body(x_ref, filw_ref, filb_ref, fv_ref):
    fv = jnp.dot(x_ref[...], filw_ref[...], preferred_element_type=jnp.float32)
    fv_ref[...] = fv + filb_ref[...]


def _out_body(x_ref, fv_ref, death_ref, rep_ref, p_ref, w1_ref, w2_ref,
              outb_ref, rhr_ref, out_ref):
    rep = rep_ref[...]
    b = jnp.dot(fv_ref[...], rep, preferred_element_type=jnp.float32)
    d = jnp.dot(death_ref[...], rep, preferred_element_type=jnp.float32)

    tt = p_ref[0:1, :]
    mu0 = p_ref[1:2, :]
    mu1 = p_ref[2:3, :]
    isig = p_ref[3:4, :]
    w0 = p_ref[4:5, :]
    w1 = p_ref[5:6, :]
    lb = p_ref[6:7, :]
    c0 = p_ref[7:8, :]
    c1 = p_ref[8:9, :]
    tcol = p_ref[9:10, :]

    tri = jax.nn.relu(d - jnp.abs(tt - b))
    gauss = jnp.exp(((b - mu0) ** 2 + (d - mu1) ** 2) * isig)
    line = b * w0 + d * w1 + lb
    r_abs = jnp.abs(rhr_ref[0])
    l1 = jnp.abs(b - c0) + jnp.abs(d - c1)
    rh = 1.0 / (1.0 + l1) - 1.0 / (1.0 + jnp.abs(r_abs - l1))
    coord = jnp.where(tcol < 0.5, tri,
                      jnp.where(tcol < 1.5, gauss,
                                jnp.where(tcol < 2.5, line, rh)))

    out = jnp.dot(x_ref[...], w1_ref[...], preferred_element_type=jnp.float32)
    out = out + jnp.dot(coord, w2_ref[...], preferred_element_type=jnp.float32)
    out_ref[...] = out + outb_ref[...]


def _vperm16(x, idx):
    dn = lax.GatherDimensionNumbers(
        offset_dims=(), collapsed_slice_dims=(0,), start_index_map=(0,))
    return lax.gather(x, idx[:, None], dn, slice_sizes=(1,),
                      mode=lax.GatherScatterMode.PROMISE_IN_BOUNDS)


def _sc_segmax_death(fv_flat, batch_p, n_pad):
    chunk = n_pad // _NS
    pairs = chunk // 2
    dchunk = n_pad // (_NS * _NC)
    dpairs = dchunk // 2
    slc = (_G * _F) // _NS
    mesh = plsc.VectorSubcoreMesh(core_axis_name="c", subcore_axis_name="s")

    @functools.partial(
        pl.kernel, mesh=mesh,
        compiler_params=pltpu.CompilerParams(needs_layout_passes=False),
        out_type=jax.ShapeDtypeStruct((n_pad * _F,), jnp.float32),
        scratch_types=[
            pltpu.VMEM((chunk * _F,), jnp.float32),
            pltpu.VMEM((chunk,), jnp.int32),
            pltpu.VMEM((chunk * _F,), jnp.int32),
            pltpu.VMEM((_GP * _F,), jnp.float32),
            pltpu.VMEM((_NS, slc), jnp.float32),
            pltpu.VMEM((_GP * _F,), jnp.float32),
            pltpu.VMEM((dchunk * _F,), jnp.float32),
            pltpu.VMEM_SHARED((_NS, _GP * _F), jnp.float32),
            pltpu.VMEM_SHARED((_G * _F,), jnp.float32),
        ],
    )
    def sc_kernel(fv_hbm, batch_hbm, death_hbm,
                  fvv, bv, idxv, localv, stagev, finalv, deathv,
                  sh_loc, sh_fin):
        c = lax.axis_index("c")
        s = lax.axis_index("s")
        wid = s * _NC + c

        i16 = lax.iota(jnp.int32, 16)
        node_off = lax.shift_right_logical(i16, 3)
        feat = lax.bitwise_and(i16, 7)
        perm = lax.bitwise_xor(i16, 8)

        neg = jnp.full((16,), _NEG, jnp.float32)

        def init_body(j, _):
            localv[pl.ds(j * 16, 16)] = neg
            return _
        lax.fori_loop(0, (_GP * _F) // 16, init_body, None)

        pltpu.sync_copy(fv_hbm.at[pl.ds(s * chunk * _F, chunk * _F)], fvv)
        pltpu.sync_copy(batch_hbm.at[pl.ds(s * chunk, chunk)], bv)

        def scan_body(i, _):
            base = i * 16
            v = fvv[pl.ds(base, 16)]
            g = plsc.load_gather(bv, [node_off + 2 * i])
            idx = g * 8 + feat
            idxv[pl.ds(base, 16)] = idx
            old = plsc.load_gather(localv, [idx])
            new = jnp.maximum(old, v)
            idx_sw = _vperm16(idx, perm)
            new_sw = _vperm16(new, perm)
            new = jnp.where(idx == idx_sw, jnp.maximum(new, new_sw), new)
            plsc.store_scatter(localv, [idx], new)
            return _
        lax.fori_loop(0, pairs, scan_body, None)

        pltpu.sync_copy(localv, sh_loc.at[s])
        plsc.subcore_barrier()

        pltpu.sync_copy(sh_loc.at[:, pl.ds(s * slc, slc)], stagev)

        def comb_body(j, _):
            m = stagev[0, pl.ds(j * 16, 16)]
            def inner(t, m):
                return jnp.maximum(m, stagev[t, pl.ds(j * 16, 16)])
            m = lax.fori_loop(1, _NS, inner, m)
            finalv[pl.ds(j * 16, 16)] = m
            return _
        lax.fori_loop(0, slc // 16, comb_body, None)
        pltpu.sync_copy(finalv.at[pl.ds(0, slc)],
                        sh_fin.at[pl.ds(s * slc, slc)])
        plsc.subcore_barrier()

        pltpu.sync_copy(sh_fin, finalv.at[pl.ds(0, _G * _F)])
        zero = jnp.zeros((16,), jnp.float32)

        def zpad_body(j, _):
            finalv[pl.ds(_G * _F + j * 16, 16)] = zero
            return _
        lax.fori_loop(0, ((_GP - _G) * _F) // 16, zpad_body, None)

        ibase = c * dchunk * _F

        def death_body(j, _):
            idx = idxv[pl.ds(ibase + j * 16, 16)]
            deathv[pl.ds(j * 16, 16)] = plsc.load_gather(finalv, [idx])
            return _
        lax.fori_loop(0, dpairs, death_body, None)

        pltpu.sync_copy(deathv,
                        death_hbm.at[pl.ds(wid * dchunk * _F, dchunk * _F)])

    return sc_kernel(fv_flat, batch_p)


def kernel(x, batch, fil_W, fil_b, tri_t, gauss_mu, gauss_sigma,
           line_W, line_b, rh_c, rh_r, out_W, out_b):
    n = x.shape[0]
    nblk = (n + _BLK - 1) // _BLK
    n_pad = nblk * _BLK
    batch_p = jnp.pad(batch, (0, n_pad - n), constant_values=_G)

    fv = pl.pallas_call(
        _fv_body,
        grid=(nblk,),
        in_specs=[
            pl.BlockSpec((_BLK, _D), lambda i: (i, 0)),
            pl.BlockSpec((_D, _F), lambda i: (0, 0)),
            pl.BlockSpec((1, _F), lambda i: (0, 0)),
        ],
        out_specs=pl.BlockSpec((_BLK, _F), lambda i: (i, 0)),
        out_shape=jax.ShapeDtypeStruct((n_pad, _F), jnp.float32),
    )(x, fil_W, fil_b.reshape(1, _F))

    death = _sc_segmax_death(fv.reshape(-1), batch_p, n_pad)
    death = death.reshape(n_pad, _F)

    nc = 12 * _F
    j = jnp.arange(nc, dtype=jnp.int32) % 12
    k3 = jnp.clip(j, 0, 2)
    kg = jnp.clip(j - 3, 0, 2)
    kl = jnp.clip(j - 6, 0, 2)
    kr = jnp.clip(j - 9, 0, 2)
    tcol = ((j >= 3).astype(jnp.float32) + (j >= 6).astype(jnp.float32)
            + (j >= 9).astype(jnp.float32))
    isig = jnp.where(tcol == 1.0,
                     -1.0 / (2.0 * gauss_sigma[kg] ** 2), -1.0)
    params = jnp.stack([
        tri_t[k3], gauss_mu[kg, 0], gauss_mu[kg, 1], isig,
        line_W[0, kl], line_W[1, kl], line_b[kl],
        rh_c[kr, 0], rh_c[kr, 1], tcol,
    ])
    params = jnp.pad(params, ((0, 6), (0, 0)))
    rep = (jnp.arange(nc, dtype=jnp.int32)[None, :] // 12
           == jnp.arange(_F, dtype=jnp.int32)[:, None]).astype(jnp.float32)

    smem = pl.BlockSpec(memory_space=pltpu.SMEM)
    out_p = pl.pallas_call(
        _out_body,
        grid=(nblk,),
        in_specs=[
            pl.BlockSpec((_BLK, _D), lambda i: (i, 0)),
            pl.BlockSpec((_BLK, _F), lambda i: (i, 0)),
            pl.BlockSpec((_BLK, _F), lambda i: (i, 0)),
            pl.BlockSpec((_F, nc), lambda i: (0, 0)),
            pl.BlockSpec((16, nc), lambda i: (0, 0)),
            pl.BlockSpec((_D, _D), lambda i: (0, 0)),
            pl.BlockSpec((nc, _D), lambda i: (0, 0)),
            pl.BlockSpec((1, _D), lambda i: (0, 0)),
            smem,
        ],
        out_specs=pl.BlockSpec((_BLK, _D), lambda i: (i, 0)),
        out_shape=jax.ShapeDtypeStruct((n, _D), jnp.float32),
    )(x, fv, death, rep, params, out_W[:_D], out_W[_D:],
      out_b.reshape(1, _D), rh_r)

    return out_p


# speedup vs baseline: 1.3465x; 1.0016x over previous
import functools

import jax
import jax.numpy as jnp
from jax import lax
from jax.experimental import pallas as pl
from jax.experimental.pallas import tpu as pltpu
from jax.experimental.pallas import tpu_sc as plsc

_G = 512
_F = 8
_D = 256
_BLK = 512
_NEG = -3.4e38

_NS = 16
_NC = 2
_GP = 520


def _fv_body(x_ref, filw_ref, filb_ref, fv_ref):
    fv = jnp.dot(x_ref[...], filw_ref[...], preferred_element_type=jnp.float32)
    fv_ref[...] = fv + filb_ref[...]


def _out_body(x_ref, fv_ref, death_ref, rep_ref, p_ref, w1_ref, w2_ref,
              outb_ref, rhr_ref, out_ref):
    rep = rep_ref[...]
    b = jnp.dot(fv_ref[...], rep, preferred_element_type=jnp.float32)
    d = jnp.dot(death_ref[...], rep, preferred_element_type=jnp.float32)

    tt = p_ref[0:1, :]
    mu0 = p_ref[1:2, :]
    mu1 = p_ref[2:3, :]
    isig = p_ref[3:4, :]
    w0 = p_ref[4:5, :]
    w1 = p_ref[5:6, :]
    lb = p_ref[6:7, :]
    c0 = p_ref[7:8, :]
    c1 = p_ref[8:9, :]
    tcol = p_ref[9:10, :]

    tri = jax.nn.relu(d - jnp.abs(tt - b))
    gauss = jnp.exp(((b - mu0) ** 2 + (d - mu1) ** 2) * isig)
    line = b * w0 + d * w1 + lb
    r_abs = jnp.abs(rhr_ref[0])
    l1 = jnp.abs(b - c0) + jnp.abs(d - c1)
    b1 = 1.0 + l1
    b2 = 1.0 + jnp.abs(r_abs - l1)
    rh = (b2 - b1) / (b1 * b2)
    coord = jnp.where(tcol < 0.5, tri,
                      jnp.where(tcol < 1.5, gauss,
                                jnp.where(tcol < 2.5, line, rh)))

    out = jnp.dot(x_ref[...], w1_ref[...], preferred_element_type=jnp.float32)
    out = out + jnp.dot(coord, w2_ref[...], preferred_element_type=jnp.float32)
    out_ref[...] = out + outb_ref[...]


def _vperm16(x, idx):
    dn = lax.GatherDimensionNumbers(
        offset_dims=(), collapsed_slice_dims=(0,), start_index_map=(0,))
    return lax.gather(x, idx[:, None], dn, slice_sizes=(1,),
                      mode=lax.GatherScatterMode.PROMISE_IN_BOUNDS)


def _sc_segmax_death(fv_flat, batch_p, n_pad):
    chunk = n_pad // _NS
    pairs = chunk // 2
    nstr = 4
    spairs = pairs // nstr
    dchunk = n_pad // (_NS * _NC)
    dpairs = dchunk // 2
    dstr = 4
    sdpairs = dpairs // dstr
    slc = (_G * _F) // _NS
    mesh = plsc.VectorSubcoreMesh(core_axis_name="c", subcore_axis_name="s")

    @functools.partial(
        pl.kernel, mesh=mesh,
        compiler_params=pltpu.CompilerParams(needs_layout_passes=False),
        out_type=jax.ShapeDtypeStruct((n_pad * _F,), jnp.float32),
        scratch_types=[
            pltpu.VMEM((chunk * _F,), jnp.float32),
            pltpu.VMEM((chunk,), jnp.int32),
            pltpu.VMEM((chunk * _F,), jnp.int32),
            pltpu.VMEM((_GP * _F,), jnp.float32),
            pltpu.VMEM((_GP * _F,), jnp.float32),
            pltpu.VMEM((_GP * _F,), jnp.float32),
            pltpu.VMEM((_GP * _F,), jnp.float32),
            pltpu.VMEM((_NS, slc), jnp.float32),
            pltpu.VMEM((_GP * _F,), jnp.float32),
            pltpu.VMEM((dchunk * _F,), jnp.float32),
            pltpu.VMEM_SHARED((_NS, _GP * _F), jnp.float32),
            pltpu.VMEM_SHARED((_G * _F,), jnp.float32),
        ],
    )
    def sc_kernel(fv_hbm, batch_hbm, death_hbm,
                  fvv, bv, idxv, localv, localv1, localv2, localv3,
                  stagev, finalv, deathv, sh_loc, sh_fin):
        locals_ = (localv, localv1, localv2, localv3)
        c = lax.axis_index("c")
        s = lax.axis_index("s")
        wid = s * _NC + c

        i16 = lax.iota(jnp.int32, 16)
        node_off = lax.shift_right_logical(i16, 3)
        feat = lax.bitwise_and(i16, 7)
        perm = lax.bitwise_xor(i16, 8)

        neg = jnp.full((16,), _NEG, jnp.float32)

        def init_body(j, _):
            for lv in locals_:
                lv[pl.ds(j * 16, 16)] = neg
            return _
        lax.fori_loop(0, (_GP * _F) // 16, init_body, None)

        pltpu.sync_copy(fv_hbm.at[pl.ds(s * chunk * _F, chunk * _F)], fvv)
        pltpu.sync_copy(batch_hbm.at[pl.ds(s * chunk, chunk)], bv)

        def scan_body(i, _):
            for k in range(nstr):
                p = k * spairs + i
                base = p * 16
                v = fvv[pl.ds(base, 16)]
                g = plsc.load_gather(bv, [node_off + 2 * p])
                idx = g * 8 + feat
                idxv[pl.ds(base, 16)] = idx
                lv = locals_[k]
                old = plsc.load_gather(lv, [idx])
                new = jnp.maximum(old, v)
                idx_sw = _vperm16(idx, perm)
                new_sw = _vperm16(new, perm)
                new = jnp.where(idx == idx_sw, jnp.maximum(new, new_sw), new)
                plsc.store_scatter(lv, [idx], new)
            return _
        lax.fori_loop(0, spairs, scan_body, None)

        def merge_body(j, _):
            m = jnp.maximum(
                jnp.maximum(localv[pl.ds(j * 16, 16)],
                            localv1[pl.ds(j * 16, 16)]),
                jnp.maximum(localv2[pl.ds(j * 16, 16)],
                            localv3[pl.ds(j * 16, 16)]))
            localv[pl.ds(j * 16, 16)] = m
            return _
        lax.fori_loop(0, (_GP * _F) // 16, merge_body, None)

        pltpu.sync_copy(localv, sh_loc.at[s])
        plsc.subcore_barrier()

        pltpu.sync_copy(sh_loc.at[:, pl.ds(s * slc, slc)], stagev)

        def comb_body(j, _):
            m = stagev[0, pl.ds(j * 16, 16)]
            def inner(t, m):
                return jnp.maximum(m, stagev[t, pl.ds(j * 16, 16)])
            m = lax.fori_loop(1, _NS, inner, m)
            finalv[pl.ds(j * 16, 16)] = m
            return _
        lax.fori_loop(0, slc // 16, comb_body, None)
        pltpu.sync_copy(finalv.at[pl.ds(0, slc)],
                        sh_fin.at[pl.ds(s * slc, slc)])
        plsc.subcore_barrier()

        pltpu.sync_copy(sh_fin, finalv.at[pl.ds(0, _G * _F)])
        zero = jnp.zeros((16,), jnp.float32)

        def zpad_body(j, _):
            finalv[pl.ds(_G * _F + j * 16, 16)] = zero
            return _
        lax.fori_loop(0, ((_GP - _G) * _F) // 16, zpad_body, None)

        ibase = c * dchunk * _F

        def death_body(j, _):
            for k in range(dstr):
                off = (k * sdpairs + j) * 16
                idx = idxv[pl.ds(ibase + off, 16)]
                deathv[pl.ds(off, 16)] = plsc.load_gather(finalv, [idx])
            return _
        lax.fori_loop(0, sdpairs, death_body, None)

        pltpu.sync_copy(deathv,
                        death_hbm.at[pl.ds(wid * dchunk * _F, dchunk * _F)])

    return sc_kernel(fv_flat, batch_p)


def kernel(x, batch, fil_W, fil_b, tri_t, gauss_mu, gauss_sigma,
           line_W, line_b, rh_c, rh_r, out_W, out_b):
    n = x.shape[0]
    nblk = (n + _BLK - 1) // _BLK
    n_pad = nblk * _BLK
    batch_p = jnp.pad(batch, (0, n_pad - n), constant_values=_G)

    fv = pl.pallas_call(
        _fv_body,
        grid=(nblk,),
        in_specs=[
            pl.BlockSpec((_BLK, _D), lambda i: (i, 0)),
            pl.BlockSpec((_D, _F), lambda i: (0, 0)),
            pl.BlockSpec((1, _F), lambda i: (0, 0)),
        ],
        out_specs=pl.BlockSpec((_BLK, _F), lambda i: (i, 0)),
        out_shape=jax.ShapeDtypeStruct((n_pad, _F), jnp.float32),
    )(x, fil_W, fil_b.reshape(1, _F))

    death = _sc_segmax_death(fv.reshape(-1), batch_p, n_pad)
    death = death.reshape(n_pad, _F)

    nc = 12 * _F
    j = jnp.arange(nc, dtype=jnp.int32) % 12
    k3 = jnp.clip(j, 0, 2)
    kg = jnp.clip(j - 3, 0, 2)
    kl = jnp.clip(j - 6, 0, 2)
    kr = jnp.clip(j - 9, 0, 2)
    tcol = ((j >= 3).astype(jnp.float32) + (j >= 6).astype(jnp.float32)
            + (j >= 9).astype(jnp.float32))
    isig = jnp.where(tcol == 1.0,
                     -1.0 / (2.0 * gauss_sigma[kg] ** 2), -1.0)
    params = jnp.stack([
        tri_t[k3], gauss_mu[kg, 0], gauss_mu[kg, 1], isig,
        line_W[0, kl], line_W[1, kl], line_b[kl],
        rh_c[kr, 0], rh_c[kr, 1], tcol,
    ])
    params = jnp.pad(params, ((0, 6), (0, 0)))
    rep = (jnp.arange(nc, dtype=jnp.int32)[None, :] // 12
           == jnp.arange(_F, dtype=jnp.int32)[:, None]).astype(jnp.float32)

    smem = pl.BlockSpec(memory_space=pltpu.SMEM)
    out_p = pl.pallas_call(
        _out_body,
        grid=(nblk,),
        in_specs=[
            pl.BlockSpec((_BLK, _D), lambda i: (i, 0)),
            pl.BlockSpec((_BLK, _F), lambda i: (i, 0)),
            pl.BlockSpec((_BLK, _F), lambda i: (i, 0)),
            pl.BlockSpec((_F, nc), lambda i: (0, 0)),
            pl.BlockSpec((16, nc), lambda i: (0, 0)),
            pl.BlockSpec((_D, _D), lambda i: (0, 0)),
            pl.BlockSpec((nc, _D), lambda i: (0, 0)),
            pl.BlockSpec((1, _D), lambda i: (0, 0)),
            smem,
        ],
        out_specs=pl.BlockSpec((_BLK, _D), lambda i: (i, 0)),
        out_shape=jax.ShapeDtypeStruct((n, _D), jnp.float32),
    )(x, fv, death, rep, params, out_W[:_D], out_W[_D:],
      out_b.reshape(1, _D), rh_r)

    return out_p
